# baseline (speedup 1.0000x reference)
import functools

import jax
import jax.numpy as jnp
from jax import lax
from jax.experimental import pallas as pl
from jax.experimental.pallas import tpu as pltpu
from jax.experimental.pallas import tpu_sc as plsc

N_NODES = 10000
N_EDGES = 320000
D_IN = 128
D_OUT = 128

NC = 2
NS = 16
NW = NC * NS
E_PER_W = N_EDGES // NW
CHUNK = 125
NCHUNK = E_PER_W // CHUNK
N_PAD = 10112
ROWS_PER_TILE = N_PAD // NS
DEG_W = 16
NBUF = 8
PF = 4


@functools.partial(
    pl.kernel,
    out_type=(
        jax.ShapeDtypeStruct((NC, N_PAD, D_IN), jnp.bfloat16),
        jax.ShapeDtypeStruct((NC, N_PAD, DEG_W), jnp.bfloat16),
    ),
    mesh=plsc.VectorSubcoreMesh(core_axis_name="c", subcore_axis_name="s"),
    compiler_params=pltpu.CompilerParams(use_tc_tiling_on_sc=False),
    scratch_types=[
        pltpu.VMEM((NCHUNK, CHUNK), jnp.int32),
        pltpu.VMEM((NCHUNK, CHUNK), jnp.int32),
        pltpu.VMEM((NBUF, CHUNK, D_IN), jnp.bfloat16),
        pltpu.VMEM((CHUNK, DEG_W), jnp.bfloat16),
        pltpu.VMEM_SHARED((N_PAD, D_IN), jnp.bfloat16),
        pltpu.VMEM_SHARED((N_PAD, DEG_W), jnp.bfloat16),
        [pltpu.SemaphoreType.DMA] * NBUF,
        [pltpu.SemaphoreType.DMA] * NBUF,
        pltpu.SemaphoreType.DMA,
    ],
)
def _sc_aggregate(hb_hbm, edges_hbm, ones_hbm, zacc_hbm, zdeg_hbm,
                  acc_out, deg_out,
                  idx_s, idx_d, rows, ones_v, acc_sh, deg_sh,
                  gsems, ssems, dsem):
    cid = lax.axis_index("c")
    sid = lax.axis_index("s")
    wid = sid * NC + cid

    pltpu.sync_copy(edges_hbm.at[0, wid], idx_s)
    pltpu.sync_copy(edges_hbm.at[1, wid], idx_d)
    pltpu.sync_copy(ones_hbm, ones_v)

    base = sid * ROWS_PER_TILE
    pltpu.sync_copy(zacc_hbm, acc_sh.at[pl.ds(base, ROWS_PER_TILE)])
    pltpu.sync_copy(zdeg_hbm, deg_sh.at[pl.ds(base, ROWS_PER_TILE)])
    plsc.subcore_barrier()

    def start_gather(c, b):
        pltpu.async_copy(hb_hbm.at[idx_s.at[c]], rows.at[b], gsems[b])

    def wait_gather(c, b):
        pltpu.make_async_copy(hb_hbm.at[idx_s.at[c]],
                              rows.at[b], gsems[b]).wait()

    def start_scatter(c, b):
        pltpu.async_copy(rows.at[b], acc_sh.at[idx_d.at[c]], ssems[b],
                         add=True)

    def wait_scatter(c, b):
        pltpu.make_async_copy(rows.at[b], acc_sh.at[idx_d.at[c]],
                              ssems[b]).wait()

    for p in range(PF):
        start_gather(p, p)

    def body(q, carry):
        for bb in range(NBUF):
            c = NBUF * q + bb
            wait_gather(c, bb)
            start_scatter(c, bb)

            pltpu.async_copy(ones_v, deg_sh.at[idx_d.at[c]], dsem, add=True)

            nb = (bb + PF) % NBUF

            @pl.when(c + PF < NCHUNK)
            def _():
                @pl.when(c >= PF)
                def _():
                    wait_scatter(c - PF, nb)

                start_gather(c + PF, nb)

        return carry

    lax.fori_loop(0, NCHUNK // NBUF, body, 0)
    for bb in range(NBUF):
        wait_scatter(NCHUNK - NBUF + bb, bb)

    def drain_deg(c, carry):
        pltpu.make_async_copy(ones_v, deg_sh.at[idx_d.at[c]], dsem).wait()
        return carry

    lax.fori_loop(0, NCHUNK, drain_deg, 0)
    plsc.subcore_barrier()

    pltpu.sync_copy(acc_sh.at[pl.ds(base, ROWS_PER_TILE)],
                    acc_out.at[cid, pl.ds(base, ROWS_PER_TILE)])
    pltpu.sync_copy(deg_sh.at[pl.ds(base, ROWS_PER_TILE)],
                    deg_out.at[cid, pl.ds(base, ROWS_PER_TILE)])


ROW_BLK = 2000


def _tc_self_body(h_ref, w1_ref, b_ref, o_ref):
    o_ref[...] = (
        jnp.dot(h_ref[...], w1_ref[...], preferred_element_type=jnp.float32)
        + b_ref[...]
    )


def _tc_neigh_body(p_ref, a_ref, d_ref, w2_ref, o_ref):
    rdeg = 1.0 / jnp.maximum(d_ref[:, 0:1], 1.0)
    o_ref[...] = (
        p_ref[...]
        + jnp.dot(a_ref[...] * rdeg, w2_ref[...],
                  preferred_element_type=jnp.float32)
    )


def _tc_self(h, w1t, b2d):
    grid = (N_NODES // ROW_BLK,)
    return pl.pallas_call(
        _tc_self_body,
        grid=grid,
        in_specs=[
            pl.BlockSpec((ROW_BLK, D_IN), lambda i: (i, 0)),
            pl.BlockSpec((D_IN, D_OUT), lambda i: (0, 0)),
            pl.BlockSpec((1, D_OUT), lambda i: (0, 0)),
        ],
        out_specs=pl.BlockSpec((ROW_BLK, D_OUT), lambda i: (i, 0)),
        out_shape=jax.ShapeDtypeStruct((N_NODES, D_OUT), jnp.float32),
    )(h, w1t, b2d)


def _tc_neigh(partial, acc_sum, deg_sum, w2t):
    grid = (N_NODES // ROW_BLK,)
    return pl.pallas_call(
        _tc_neigh_body,
        grid=grid,
        in_specs=[
            pl.BlockSpec((ROW_BLK, D_OUT), lambda i: (i, 0)),
            pl.BlockSpec((ROW_BLK, D_IN), lambda i: (i, 0)),
            pl.BlockSpec((ROW_BLK, DEG_W), lambda i: (i, 0)),
            pl.BlockSpec((D_IN, D_OUT), lambda i: (0, 0)),
        ],
        out_specs=pl.BlockSpec((ROW_BLK, D_OUT), lambda i: (i, 0)),
        out_shape=jax.ShapeDtypeStruct((N_NODES, D_OUT), jnp.float32),
    )(partial, acc_sum, deg_sum, w2t)


def kernel(h, edge_index, W, b):
    edges = edge_index.astype(jnp.int32).reshape(2, NW, NCHUNK, CHUNK)
    hb = h.astype(jnp.bfloat16)
    ones = jnp.ones((CHUNK, DEG_W), dtype=jnp.bfloat16)
    zacc = jnp.zeros((ROWS_PER_TILE, D_IN), dtype=jnp.bfloat16)
    zdeg = jnp.zeros((ROWS_PER_TILE, DEG_W), dtype=jnp.bfloat16)

    acc, deg = _sc_aggregate(hb, edges, ones, zacc, zdeg)

    w1t = W[:, :D_IN].T
    w2t = W[:, D_IN:].T
    b2d = b.reshape(1, D_OUT)
    partial = _tc_self(h, w1t, b2d)
    acc_sum = acc[0].astype(jnp.float32) + acc[1].astype(jnp.float32)
    deg_sum = deg[0].astype(jnp.float32) + deg[1].astype(jnp.float32)
    return _tc_neigh(partial, acc_sum, deg_sum, w2t)
